# baseline (speedup 1.0000x reference)
import functools

import jax
import jax.numpy as jnp
from jax import lax
from jax.experimental import pallas as pl
from jax.experimental.pallas import tpu as pltpu
from jax.experimental.pallas import tpu_sc as plsc

D = 32
NC, NS = 2, 16
NW = NC * NS
BL = 128
DT, DL = 4, 8
L = 16


@functools.partial(jax.jit, static_argnames=("t_len", "b_len"))
def _fused_lookup(x2, table, t_len, b_len):
    n_bt = b_len // BL
    assert n_bt == NW
    mesh = plsc.VectorSubcoreMesh(
        core_axis_name="c", subcore_axis_name="s", num_cores=NC, num_subcores=NS
    )

    @functools.partial(
        pl.kernel,
        out_type=jax.ShapeDtypeStruct((t_len, DT, n_bt, DL, BL), jnp.float32),
        mesh=mesh,
        scratch_types=[
            pltpu.VMEM((t_len, BL), jnp.int32),
            pltpu.VMEM((2, BL, D), jnp.float32),
            pltpu.VMEM((2, DT, DL, BL), jnp.float32),
            pltpu.SemaphoreType.DMA,
            pltpu.SemaphoreType.DMA,
            pltpu.SemaphoreType.DMA,
            pltpu.SemaphoreType.DMA,
        ],
        compiler_params=pltpu.CompilerParams(
            use_tc_tiling_on_sc=False, needs_layout_passes=False
        ),
    )
    def body(x2_hbm, tab_hbm, out_hbm, idx_v, rows_v, tile_v, g0, g1, w0, w1):
        gsem = [g0, g1]
        wsem = [w0, w1]
        w = lax.axis_index("s") * NC + lax.axis_index("c")
        iota = lax.iota(jnp.int32, L)

        pltpu.sync_copy(x2_hbm.at[:, pl.ds(w * BL, BL)], idx_v)
        pltpu.async_copy(tab_hbm.at[idx_v.at[0]], rows_v.at[0], gsem[0])

        def half_step(t, phase):
            nt = t + 1

            @pl.when(nt < t_len)
            def _():
                pltpu.async_copy(
                    tab_hbm.at[idx_v.at[nt]], rows_v.at[1 - phase],
                    gsem[1 - phase],
                )

            pltpu.make_async_copy(
                tab_hbm.at[idx_v.at[t]], rows_v.at[phase], gsem[phase]
            ).wait()

            @pl.when(t >= 2)
            def _():
                pltpu.make_async_copy(
                    tile_v.at[phase], out_hbm.at[t, :, w], wsem[phase]
                ).wait()

            for dt in range(DT):
                for dl in range(DL):
                    col = jnp.full((L,), dt * DL + dl, jnp.int32)
                    for bc in range(BL // L):
                        row = bc * L + iota
                        vec = plsc.load_gather(
                            rows_v.at[phase], [row, col]
                        )
                        tile_v[phase, dt, dl, pl.ds(bc * L, L)] = vec

            pltpu.async_copy(tile_v.at[phase], out_hbm.at[t, :, w], wsem[phase])

        def step(t2, _):
            half_step(2 * t2, 0)
            half_step(2 * t2 + 1, 1)
            return 0

        lax.fori_loop(0, t_len // 2, step, 0)
        pltpu.make_async_copy(
            tile_v.at[0], out_hbm.at[t_len - 2, :, w], wsem[0]
        ).wait()
        pltpu.make_async_copy(
            tile_v.at[1], out_hbm.at[t_len - 1, :, w], wsem[1]
        ).wait()

    return body(x2, table)


def kernel(x, table):
    b, t = x.shape
    x2 = jnp.transpose(x)
    out5 = _fused_lookup(x2.astype(jnp.int32), table, t, b)
    return out5.transpose(2, 4, 0, 1, 3).reshape(b, t, D)


# speedup vs baseline: 1.0637x; 1.0637x over previous
import functools

import jax
import jax.numpy as jnp
from jax import lax
from jax.experimental import pallas as pl
from jax.experimental.pallas import tpu as pltpu
from jax.experimental.pallas import tpu_sc as plsc

D = 32
NC, NS = 2, 16
NW = NC * NS
BL = 128
DT, DL = 4, 8
L = 16
TB = 4


@functools.partial(jax.jit, static_argnames=("t_len", "b_len"))
def _fused_lookup(x2, table, t_len, b_len):
    n_bt = b_len // BL
    assert n_bt == NW and t_len % (2 * TB) == 0
    mesh = plsc.VectorSubcoreMesh(
        core_axis_name="c", subcore_axis_name="s", num_cores=NC, num_subcores=NS
    )

    @functools.partial(
        pl.kernel,
        out_type=jax.ShapeDtypeStruct((t_len, DT, n_bt, DL, BL), jnp.float32),
        mesh=mesh,
        scratch_types=[
            pltpu.VMEM((t_len, BL), jnp.int32),
            pltpu.VMEM((2, TB * BL, D), jnp.float32),
            pltpu.VMEM((2, TB, DT, DL, BL), jnp.float32),
            pltpu.SemaphoreType.DMA,
            pltpu.SemaphoreType.DMA,
            pltpu.SemaphoreType.DMA,
            pltpu.SemaphoreType.DMA,
        ],
        compiler_params=pltpu.CompilerParams(
            use_tc_tiling_on_sc=False, needs_layout_passes=False
        ),
    )
    def body(x2_hbm, tab_hbm, out_hbm, idx_v, rows_v, tile_v, g0, g1, w0, w1):
        gsem = [g0, g1]
        wsem = [w0, w1]
        w = lax.axis_index("s") * NC + lax.axis_index("c")
        iota = lax.iota(jnp.int32, L)

        pltpu.sync_copy(x2_hbm.at[:, pl.ds(w * BL, BL)], idx_v)

        def fire(step, phase):
            for j in range(TB):
                pltpu.async_copy(
                    tab_hbm.at[idx_v.at[step * TB + j]],
                    rows_v.at[phase, pl.ds(j * BL, BL)],
                    gsem[phase],
                )

        def drain_gathers(step, phase):
            for j in range(TB):
                pltpu.make_async_copy(
                    tab_hbm.at[idx_v.at[step * TB + j]],
                    rows_v.at[phase, pl.ds(j * BL, BL)],
                    gsem[phase],
                ).wait()

        fire(0, 0)

        def half_step(step, phase):
            @pl.when(step + 1 < t_len // TB)
            def _():
                fire(step + 1, 1 - phase)

            drain_gathers(step, phase)

            @pl.when(step >= 2)
            def _():
                pltpu.make_async_copy(
                    tile_v.at[phase],
                    out_hbm.at[pl.ds(step * TB, TB), :, w],
                    wsem[phase],
                ).wait()

            def tpose(tl, _):
                rbase = iota + tl * BL

                for dt in range(DT):
                    for dl in range(DL):
                        col = jnp.full((L,), dt * DL + dl, jnp.int32)
                        for bc in range(BL // L):
                            row = rbase + bc * L
                            vec = plsc.load_gather(
                                rows_v.at[phase], [row, col]
                            )
                            tile_v[phase, tl, dt, dl, pl.ds(bc * L, L)] = vec
                return 0

            lax.fori_loop(0, TB, tpose, 0)

            pltpu.async_copy(
                tile_v.at[phase],
                out_hbm.at[pl.ds(step * TB, TB), :, w],
                wsem[phase],
            )

        def step_fn(i, _):
            half_step(2 * i, 0)
            half_step(2 * i + 1, 1)
            return 0

        n_steps = t_len // TB
        lax.fori_loop(0, n_steps // 2, step_fn, 0)
        pltpu.make_async_copy(
            tile_v.at[0], out_hbm.at[pl.ds(t_len - 2 * TB, TB), :, w], wsem[0]
        ).wait()
        pltpu.make_async_copy(
            tile_v.at[1], out_hbm.at[pl.ds(t_len - TB, TB), :, w], wsem[1]
        ).wait()

    return body(x2, table)


def kernel(x, table):
    b, t = x.shape
    x2 = jnp.transpose(x)
    out5 = _fused_lookup(x2.astype(jnp.int32), table, t, b)
    return out5.transpose(2, 4, 0, 1, 3).reshape(b, t, D)


# speedup vs baseline: 1.1942x; 1.1226x over previous
import functools

import jax
import jax.numpy as jnp
from jax import lax
from jax.experimental import pallas as pl
from jax.experimental.pallas import tpu as pltpu
from jax.experimental.pallas import tpu_sc as plsc

D = 32
NC, NS = 2, 16
NW = NC * NS
BL = 128
DT, DL = 4, 8
L = 16
TB = 4


@functools.partial(jax.jit, static_argnames=("t_len", "b_len"))
def _fused_lookup(x2, table, t_len, b_len):
    n_bt = b_len // BL
    assert n_bt == NW and t_len % (2 * TB) == 0
    mesh = plsc.VectorSubcoreMesh(
        core_axis_name="c", subcore_axis_name="s", num_cores=NC, num_subcores=NS
    )

    @functools.partial(
        pl.kernel,
        out_type=jax.ShapeDtypeStruct((t_len, DT, n_bt, DL, BL), jnp.float32),
        mesh=mesh,
        scratch_types=[
            pltpu.VMEM((t_len, BL), jnp.int32),
            pltpu.VMEM((2, TB * BL, D), jnp.float32),
            pltpu.VMEM((2, TB, DT, DL, BL), jnp.float32),
            pltpu.SemaphoreType.DMA,
            pltpu.SemaphoreType.DMA,
            pltpu.SemaphoreType.DMA,
            pltpu.SemaphoreType.DMA,
        ],
        compiler_params=pltpu.CompilerParams(
            use_tc_tiling_on_sc=False, needs_layout_passes=False
        ),
    )
    def body(x2_hbm, tab_hbm, out_hbm, idx_v, rows_v, tile_v, g0, g1, w0, w1):
        gsem = [g0, g1]
        wsem = [w0, w1]
        w = lax.axis_index("s") * NC + lax.axis_index("c")
        iota = lax.iota(jnp.int32, L)

        pltpu.sync_copy(x2_hbm.at[:, pl.ds(w * BL, BL)], idx_v)

        def fire(step, phase):
            for j in range(TB):
                pltpu.async_copy(
                    tab_hbm.at[idx_v.at[step * TB + j]],
                    rows_v.at[phase, pl.ds(j * BL, BL)],
                    gsem[phase],
                )

        def drain_gathers(step, phase):
            for j in range(TB):
                pltpu.make_async_copy(
                    tab_hbm.at[idx_v.at[step * TB + j]],
                    rows_v.at[phase, pl.ds(j * BL, BL)],
                    gsem[phase],
                ).wait()

        fire(0, 0)

        def half_step(step, phase):
            @pl.when(step + 1 < t_len // TB)
            def _():
                fire(step + 1, 1 - phase)

            drain_gathers(step, phase)

            @pl.when(step >= 2)
            def _():
                pltpu.make_async_copy(
                    tile_v.at[phase],
                    out_hbm.at[pl.ds(step * TB, TB), :, w],
                    wsem[phase],
                ).wait()

            def tpose(tl, _):
                rbase = iota + tl * BL

                for dt in range(DT):
                    for dl in range(DL):
                        col = jnp.full((L,), dt * DL + dl, jnp.int32)
                        vecs = [
                            plsc.load_gather(
                                rows_v.at[phase], [rbase + bc * L, col]
                            )
                            for bc in range(BL // L)
                        ]
                        for bc in range(BL // L):
                            tile_v[phase, tl, dt, dl, pl.ds(bc * L, L)] = (
                                vecs[bc]
                            )
                return 0

            lax.fori_loop(0, TB, tpose, 0)

            pltpu.async_copy(
                tile_v.at[phase],
                out_hbm.at[pl.ds(step * TB, TB), :, w],
                wsem[phase],
            )

        def step_fn(i, _):
            half_step(2 * i, 0)
            half_step(2 * i + 1, 1)
            return 0

        n_steps = t_len // TB
        lax.fori_loop(0, n_steps // 2, step_fn, 0)
        pltpu.make_async_copy(
            tile_v.at[0], out_hbm.at[pl.ds(t_len - 2 * TB, TB), :, w], wsem[0]
        ).wait()
        pltpu.make_async_copy(
            tile_v.at[1], out_hbm.at[pl.ds(t_len - TB, TB), :, w], wsem[1]
        ).wait()

    return body(x2, table)


def kernel(x, table):
    b, t = x.shape
    x2 = jnp.transpose(x)
    out5 = _fused_lookup(x2.astype(jnp.int32), table, t, b)
    return out5.transpose(2, 4, 0, 1, 3).reshape(b, t, D)


# speedup vs baseline: 1.1954x; 1.0010x over previous
import functools

import jax
import jax.numpy as jnp
from jax import lax
from jax.experimental import pallas as pl
from jax.experimental.pallas import tpu as pltpu
from jax.experimental.pallas import tpu_sc as plsc

D = 32
NC, NS = 2, 16
NW = NC * NS
BL = 128
DT, DL = 4, 8
L = 16
TB = 5


@functools.partial(jax.jit, static_argnames=("t_len", "b_len"))
def _fused_lookup(xw, table, t_len, b_len):
    n_bt = b_len // BL
    assert n_bt == NW and t_len % (2 * TB) == 0
    per_w = t_len * BL
    mesh = plsc.VectorSubcoreMesh(
        core_axis_name="c", subcore_axis_name="s", num_cores=NC, num_subcores=NS
    )

    @functools.partial(
        pl.kernel,
        out_type=jax.ShapeDtypeStruct((t_len, DT, n_bt, DL, BL), jnp.float32),
        mesh=mesh,
        scratch_types=[
            pltpu.VMEM((per_w,), jnp.int32),
            pltpu.VMEM((2, TB * BL, D), jnp.float32),
            pltpu.VMEM((2, TB, DT, DL, BL), jnp.float32),
            pltpu.SemaphoreType.DMA,
            pltpu.SemaphoreType.DMA,
            pltpu.SemaphoreType.DMA,
            pltpu.SemaphoreType.DMA,
        ],
        compiler_params=pltpu.CompilerParams(
            use_tc_tiling_on_sc=False, needs_layout_passes=False
        ),
    )
    def body(xw_hbm, tab_hbm, out_hbm, idx_v, rows_v, tile_v, g0, g1, w0, w1):
        gsem = [g0, g1]
        wsem = [w0, w1]
        w = lax.axis_index("s") * NC + lax.axis_index("c")
        iota = lax.iota(jnp.int32, L)

        pltpu.sync_copy(xw_hbm.at[w], idx_v)

        def fire(step, phase):
            pltpu.async_copy(
                tab_hbm.at[idx_v.at[pl.ds(step * TB * BL, TB * BL)]],
                rows_v.at[phase],
                gsem[phase],
            )

        def drain(step, phase):
            pltpu.make_async_copy(
                tab_hbm.at[idx_v.at[pl.ds(step * TB * BL, TB * BL)]],
                rows_v.at[phase],
                gsem[phase],
            ).wait()

        fire(0, 0)

        def half_step(step, phase):
            @pl.when(step + 1 < t_len // TB)
            def _():
                fire(step + 1, 1 - phase)

            drain(step, phase)

            @pl.when(step >= 2)
            def _():
                pltpu.make_async_copy(
                    tile_v.at[phase],
                    out_hbm.at[pl.ds(step * TB, TB), :, w],
                    wsem[phase],
                ).wait()

            def tpose(tl, _):
                rbase = iota + tl * BL

                for dt in range(DT):
                    for dl in range(DL):
                        col = jnp.full((L,), dt * DL + dl, jnp.int32)
                        vecs = [
                            plsc.load_gather(
                                rows_v.at[phase], [rbase + bc * L, col]
                            )
                            for bc in range(BL // L)
                        ]
                        for bc in range(BL // L):
                            tile_v[phase, tl, dt, dl, pl.ds(bc * L, L)] = (
                                vecs[bc]
                            )
                return 0

            lax.fori_loop(0, TB, tpose, 0)

            pltpu.async_copy(
                tile_v.at[phase],
                out_hbm.at[pl.ds(step * TB, TB), :, w],
                wsem[phase],
            )

        def step_fn(i, _):
            half_step(2 * i, 0)
            half_step(2 * i + 1, 1)
            return 0

        n_steps = t_len // TB
        lax.fori_loop(0, n_steps // 2, step_fn, 0)
        pltpu.make_async_copy(
            tile_v.at[0], out_hbm.at[pl.ds(t_len - 2 * TB, TB), :, w], wsem[0]
        ).wait()
        pltpu.make_async_copy(
            tile_v.at[1], out_hbm.at[pl.ds(t_len - TB, TB), :, w], wsem[1]
        ).wait()

    return body(xw, table)


def kernel(x, table):
    b, t = x.shape
    x2 = jnp.transpose(x)
    xw = (
        x2.astype(jnp.int32)
        .reshape(t, NW, BL)
        .transpose(1, 0, 2)
        .reshape(NW, t * BL)
    )
    out5 = _fused_lookup(xw, table, t, b)
    return out5.transpose(2, 4, 0, 1, 3).reshape(b, t, D)


# speedup vs baseline: 1.5244x; 1.2752x over previous
import functools

import jax
import jax.numpy as jnp
from jax import lax
from jax.experimental import pallas as pl
from jax.experimental.pallas import tpu as pltpu
from jax.experimental.pallas import tpu_sc as plsc

D = 32
NC, NS = 2, 16
NW = NC * NS
BL = 128
DT, DL = 4, 8
L = 16
TB = 5


@functools.partial(jax.jit, static_argnames=("t_len", "b_len"))
def _fused_lookup(xw, table, t_len, b_len):
    n_bt = b_len // BL
    assert n_bt == NW and t_len % (2 * TB) == 0
    per_w = t_len * BL
    mesh = plsc.VectorSubcoreMesh(
        core_axis_name="c", subcore_axis_name="s", num_cores=NC, num_subcores=NS
    )

    @functools.partial(
        pl.kernel,
        out_type=jax.ShapeDtypeStruct((t_len, DT, n_bt, DL, BL), jnp.float32),
        mesh=mesh,
        scratch_types=[
            pltpu.VMEM((per_w,), jnp.int32),
            pltpu.VMEM((2, TB * BL, D), jnp.float32),
            pltpu.VMEM((2, TB, DT, DL, 137), jnp.float32),
            pltpu.SemaphoreType.DMA,
            pltpu.SemaphoreType.DMA,
            pltpu.SemaphoreType.DMA,
            pltpu.SemaphoreType.DMA,
        ],
        compiler_params=pltpu.CompilerParams(
            use_tc_tiling_on_sc=False, needs_layout_passes=False
        ),
    )
    def body(xw_hbm, tab_hbm, out_hbm, idx_v, rows_v, tile_v, g0, g1, w0, w1):
        gsem = [g0, g1]
        wsem = [w0, w1]
        w = lax.axis_index("s") * NC + lax.axis_index("c")
        iota = lax.iota(jnp.int32, L)

        pltpu.sync_copy(xw_hbm.at[w], idx_v)

        def fire(step, phase):
            pltpu.async_copy(
                tab_hbm.at[idx_v.at[pl.ds(step * TB * BL, TB * BL)]],
                rows_v.at[phase],
                gsem[phase],
            )

        def drain(step, phase):
            pltpu.make_async_copy(
                tab_hbm.at[idx_v.at[pl.ds(step * TB * BL, TB * BL)]],
                rows_v.at[phase],
                gsem[phase],
            ).wait()

        fire(0, 0)

        def half_step(step, phase):
            @pl.when(step + 1 < t_len // TB)
            def _():
                fire(step + 1, 1 - phase)

            drain(step, phase)

            @pl.when(step >= 2)
            def _():
                pltpu.make_async_copy(
                    tile_v.at[phase, :, :, :, pl.ds(0, BL)],
                    out_hbm.at[pl.ds(step * TB, TB), :, w],
                    wsem[phase],
                ).wait()

            dtv = [iota // DL + h * (L // DL) for h in range(D // L)]
            dlv = [iota % DL for _ in range(D // L)]

            def tpose(tl, _):
                tref = tile_v.at[phase, tl]
                for bb in range(BL):
                    bvec = jnp.full((L,), bb, jnp.int32)
                    rr = tl * BL + bb
                    for h in range(D // L):
                        vec = rows_v[phase, rr, pl.ds(h * L, L)]
                        plsc.store_scatter(
                            tref, [dtv[h], dlv[h], bvec], vec
                        )
                return 0

            lax.fori_loop(0, TB, tpose, 0)

            pltpu.async_copy(
                tile_v.at[phase, :, :, :, pl.ds(0, BL)],
                out_hbm.at[pl.ds(step * TB, TB), :, w],
                wsem[phase],
            )

        def step_fn(i, _):
            half_step(2 * i, 0)
            half_step(2 * i + 1, 1)
            return 0

        n_steps = t_len // TB
        lax.fori_loop(0, n_steps // 2, step_fn, 0)
        pltpu.make_async_copy(
            tile_v.at[0, :, :, :, pl.ds(0, BL)],
            out_hbm.at[pl.ds(t_len - 2 * TB, TB), :, w],
            wsem[0],
        ).wait()
        pltpu.make_async_copy(
            tile_v.at[1, :, :, :, pl.ds(0, BL)],
            out_hbm.at[pl.ds(t_len - TB, TB), :, w],
            wsem[1],
        ).wait()

    return body(xw, table)


def kernel(x, table):
    b, t = x.shape
    x2 = jnp.transpose(x)
    xw = (
        x2.astype(jnp.int32)
        .reshape(t, NW, BL)
        .transpose(1, 0, 2)
        .reshape(NW, t * BL)
    )
    out5 = _fused_lookup(xw, table, t, b)
    return out5.transpose(2, 4, 0, 1, 3).reshape(b, t, D)
